# baseline (speedup 1.0000x reference)
import functools

import jax
import jax.numpy as jnp
from jax import lax
from jax.experimental import pallas as pl
from jax.experimental.pallas import tpu as pltpu
from jax.experimental.pallas import tpu_sc as plsc

B, D, T = 16, 256, 1024
N = B * T
K = 8192
COMMIT = 0.25

TBLK = 512
KBLK = 1024
NRB = N // TBLK
NKB = K // KBLK
TPB = T // TBLK


def _bf16_rn(v):
    b = lax.bitcast_convert_type(v, jnp.uint32)
    r = (b + jnp.uint32(0x7FFF) + ((b >> jnp.uint32(16)) & jnp.uint32(1))) \
        & jnp.uint32(0xFFFF0000)
    return lax.bitcast_convert_type(r, jnp.float32)


_SEG = (0, 2736, 5472, K)


def _k1_body(x_ref, e_ref, esq_ref, xsq_ref, idx_ref, loss_ref,
             minv0, mini0, minv1, mini1, minv2, mini2, acc):
    rb = pl.program_id(0)
    kb = pl.program_id(1)
    x = x_ref[0]
    e = e_ref[...]
    mm = lax.dot_general(e, x, (((1,), (0,)), ((), ())),
                         preferred_element_type=jnp.float32)
    d = (xsq_ref[...] + esq_ref[...]) - 2.0 * mm
    kio = lax.broadcasted_iota(jnp.int32, d.shape, 0) + kb * KBLK
    segs = ((minv0, mini0), (minv1, mini1), (minv2, mini2))

    for s, (mv, mi) in enumerate(segs):
        @pl.when(kb == 0)
        def _(mv=mv, mi=mi):
            mv[...] = jnp.full((1, TBLK), jnp.inf, jnp.float32)
            mi[...] = jnp.zeros((1, TBLK), jnp.int32)
        lo, hi = _SEG[s], _SEG[s + 1]
        inseg = (kio >= lo) & (kio < hi)
        ds = jnp.where(inseg, d, jnp.inf)
        bmin = jnp.min(ds, axis=0)
        bidx = jnp.min(jnp.where(ds == bmin[None, :], kio, K), axis=0)
        bmin = bmin.reshape(1, -1)
        bidx = bidx.reshape(1, -1)
        upd = bmin < mv[...]
        mi[...] = jnp.where(upd, bidx, mi[...])
        mv[...] = jnp.where(upd, bmin, mv[...])

    @pl.when(kb == NKB - 1)
    def _():
        av = _bf16_rn(minv0[...])
        ai = mini0[...]
        vex = minv0[...]
        for mv, mi in segs[1:]:
            upd = mv[...] < av
            ai = jnp.where(upd, mi[...], ai)
            vex = jnp.where(upd, mv[...], vex)
            av = jnp.where(upd, _bf16_rn(mv[...]), av)
        idx_ref[0] = ai
        part = jnp.sum(vex)
        prev = jnp.where(rb == 0, 0.0, acc[0, 0])
        acc[0, 0] = prev + part

    @pl.when((kb == NKB - 1) & (rb == NRB - 1))
    def _():
        loss_ref[...] = (acc[0, 0] * (COMMIT / (N * D))) * jnp.ones((1, 1), jnp.float32)


def _k1(x_bf, e_bf, e_sq, x_sq):
    return pl.pallas_call(
        _k1_body,
        grid=(NRB, NKB),
        in_specs=[
            pl.BlockSpec((1, D, TBLK), lambda rb, kb: (rb // TPB, 0, rb % TPB)),
            pl.BlockSpec((KBLK, D), lambda rb, kb: (kb, 0)),
            pl.BlockSpec((KBLK, 1), lambda rb, kb: (kb, 0)),
            pl.BlockSpec((1, TBLK), lambda rb, kb: (0, rb)),
        ],
        out_specs=[
            pl.BlockSpec((1, 1, TBLK), lambda rb, kb: (rb // TPB, 0, rb % TPB)),
            pl.BlockSpec((1, 1), lambda rb, kb: (0, 0)),
        ],
        out_shape=[
            jax.ShapeDtypeStruct((B, 1, T), jnp.int32),
            jax.ShapeDtypeStruct((1, 1), jnp.float32),
        ],
        scratch_shapes=[
            pltpu.VMEM((1, TBLK), jnp.float32),
            pltpu.VMEM((1, TBLK), jnp.int32),
            pltpu.VMEM((1, TBLK), jnp.float32),
            pltpu.VMEM((1, TBLK), jnp.int32),
            pltpu.VMEM((1, TBLK), jnp.float32),
            pltpu.VMEM((1, TBLK), jnp.int32),
            pltpu.SMEM((1, 1), jnp.float32),
        ],
        compiler_params=pltpu.CompilerParams(
            dimension_semantics=("arbitrary", "arbitrary")),
    )(x_bf, e_bf, e_sq, x_sq)


_NC = 2
_NS = 16
_NW = _NC * _NS
_BPW = N // _NW
_CHUNK = 256
_NCHUNK = _BPW // _CHUNK
_BINS = K // _NW
_NVREG = N // 16


def _sc_gather_hist(table32, idx_flat):
    mesh = plsc.VectorSubcoreMesh(core_axis_name="c", subcore_axis_name="s")

    @functools.partial(
        pl.kernel, mesh=mesh,
        out_type=[
            jax.ShapeDtypeStruct((N, D), jnp.float32),
            jax.ShapeDtypeStruct((K,), jnp.float32),
        ],
        scratch_types=[
            pltpu.VMEM((_CHUNK,), jnp.int32),
            pltpu.VMEM((_CHUNK, D), jnp.float32),
            pltpu.VMEM((N,), jnp.int32),
            pltpu.VMEM((_BINS,), jnp.float32),
            pltpu.SemaphoreType.DMA,
        ],
        compiler_params=pltpu.CompilerParams(needs_layout_passes=False),
    )
    def sc_k(table_hbm, idx_hbm, rows_out, counts_out,
             idx_v, rows_v, idx_all, counts_v, sem):
        wid = lax.axis_index("s") * _NC + lax.axis_index("c")
        base = wid * _BPW
        for c in range(_NCHUNK):
            pltpu.sync_copy(idx_hbm.at[pl.ds(base + c * _CHUNK, _CHUNK)], idx_v)
            pltpu.async_copy(table_hbm.at[idx_v], rows_v, sem).wait()
            pltpu.sync_copy(rows_v, rows_out.at[pl.ds(base + c * _CHUNK, _CHUNK)])
        pltpu.sync_copy(idx_hbm, idx_all)
        for j in range(_BINS // 16):
            counts_v[pl.ds(j * 16, 16)] = jnp.zeros((16,), jnp.float32)
        lo = wid * _BINS

        def body(i, carry):
            v = idx_all[pl.ds(i * 16, 16)]
            m = (v >= lo) & (v < lo + _BINS)
            vl = jnp.where(m, v - lo, 0)
            one = jnp.where(m, 1.0, 0.0).astype(jnp.float32)
            plsc.addupdate_scatter(counts_v, [vl], one)
            return carry

        lax.fori_loop(0, _NVREG, body, 0, unroll=4)
        pltpu.sync_copy(counts_v, counts_out.at[pl.ds(lo, _BINS)])

    return sc_k(table32, idx_flat)


def _k3_body(rows_ref, counts_ref, q_ref, ppl_ref):
    b = pl.program_id(0)
    q_ref[0] = rows_ref[0].T

    @pl.when(b == 0)
    def _():
        p = counts_ref[...] * (1.0 / N)
        s = jnp.sum(p * jnp.log(p + 1e-10))
        ppl_ref[...] = jnp.exp(-s) * jnp.ones((1, 1), jnp.float32)


def _k3(rows3d, counts2d):
    return pl.pallas_call(
        _k3_body,
        grid=(B,),
        in_specs=[
            pl.BlockSpec((1, T, D), lambda b: (b, 0, 0)),
            pl.BlockSpec((1, K), lambda b: (0, 0)),
        ],
        out_specs=[
            pl.BlockSpec((1, D, T), lambda b: (b, 0, 0)),
            pl.BlockSpec((1, 1), lambda b: (0, 0)),
        ],
        out_shape=[
            jax.ShapeDtypeStruct((B, D, T), jnp.float32),
            jax.ShapeDtypeStruct((1, 1), jnp.float32),
        ],
        compiler_params=pltpu.CompilerParams(
            dimension_semantics=("arbitrary",)),
    )(rows3d, counts2d)


def kernel(inputs, embedding):
    x_bf = inputs.astype(jnp.bfloat16)
    e_bf = embedding.astype(jnp.bfloat16)
    table32 = e_bf.astype(jnp.float32)
    x_sq = jnp.sum(inputs ** 2, axis=1).reshape(1, N)
    e_sq = jnp.sum(embedding ** 2, axis=1).reshape(K, 1)

    idx3, loss1 = _k1(x_bf, e_bf, e_sq, x_sq)
    idx_flat = idx3.reshape(N)
    rows, counts = _sc_gather_hist(table32, idx_flat)
    q_st, ppl = _k3(rows.reshape(B, T, D), counts.reshape(1, K))
    return (loss1.reshape(()), q_st, ppl.reshape(()), idx3.reshape(B, T))


# speedup vs baseline: 1.4679x; 1.4679x over previous
import functools

import jax
import jax.numpy as jnp
from jax import lax
from jax.experimental import pallas as pl
from jax.experimental.pallas import tpu as pltpu
from jax.experimental.pallas import tpu_sc as plsc

B, D, T = 16, 256, 1024
N = B * T
K = 8192
COMMIT = 0.25

TBLK = 512
KBLK = 1024
NRB = N // TBLK
NKB = K // KBLK
TPB = T // TBLK


def _bf16_rn(v):
    b = lax.bitcast_convert_type(v, jnp.uint32)
    r = (b + jnp.uint32(0x7FFF) + ((b >> jnp.uint32(16)) & jnp.uint32(1))) \
        & jnp.uint32(0xFFFF0000)
    return lax.bitcast_convert_type(r, jnp.float32)


_SEG = (0, 2736, 5472, K)


def _k1_body(x_ref, e_ref, esq_ref, xsq_ref, idx_ref, loss_ref,
             minv0, mini0, minv1, mini1, minv2, mini2, acc):
    rb = pl.program_id(0)
    kb = pl.program_id(1)
    x = x_ref[0]
    e = e_ref[...]
    mm = lax.dot_general(e, x, (((1,), (0,)), ((), ())),
                         preferred_element_type=jnp.float32)
    d = (xsq_ref[...] + esq_ref[...]) - 2.0 * mm
    kio = lax.broadcasted_iota(jnp.int32, d.shape, 0) + kb * KBLK
    segs = ((minv0, mini0), (minv1, mini1), (minv2, mini2))

    for s, (mv, mi) in enumerate(segs):
        @pl.when(kb == 0)
        def _(mv=mv, mi=mi):
            mv[...] = jnp.full((1, TBLK), jnp.inf, jnp.float32)
            mi[...] = jnp.zeros((1, TBLK), jnp.int32)
        lo, hi = _SEG[s], _SEG[s + 1]
        kb_lo = lo // KBLK
        kb_hi = (hi + KBLK - 1) // KBLK

        @pl.when((kb >= kb_lo) & (kb < kb_hi))
        def _(mv=mv, mi=mi, lo=lo, hi=hi):
            inseg = (kio >= lo) & (kio < hi)
            ds = jnp.where(inseg, d, jnp.inf)
            bmin = jnp.min(ds, axis=0)
            bidx = jnp.min(jnp.where(ds == bmin[None, :], kio, K), axis=0)
            bmin2 = bmin.reshape(1, -1)
            bidx2 = bidx.reshape(1, -1)
            upd = bmin2 < mv[...]
            mi[...] = jnp.where(upd, bidx2, mi[...])
            mv[...] = jnp.where(upd, bmin2, mv[...])

    @pl.when(kb == NKB - 1)
    def _():
        av = _bf16_rn(minv0[...])
        ai = mini0[...]
        vex = minv0[...]
        for mv, mi in segs[1:]:
            upd = mv[...] < av
            ai = jnp.where(upd, mi[...], ai)
            vex = jnp.where(upd, mv[...], vex)
            av = jnp.where(upd, _bf16_rn(mv[...]), av)
        idx_ref[0] = ai
        part = jnp.sum(vex)
        prev = jnp.where(rb == 0, 0.0, acc[0, 0])
        acc[0, 0] = prev + part

    @pl.when((kb == NKB - 1) & (rb == NRB - 1))
    def _():
        loss_ref[...] = (acc[0, 0] * (COMMIT / (N * D))) * jnp.ones((1, 1), jnp.float32)


def _k1(x_bf, e_bf, e_sq, x_sq):
    return pl.pallas_call(
        _k1_body,
        grid=(NRB, NKB),
        in_specs=[
            pl.BlockSpec((1, D, TBLK), lambda rb, kb: (rb // TPB, 0, rb % TPB)),
            pl.BlockSpec((KBLK, D), lambda rb, kb: (kb, 0)),
            pl.BlockSpec((KBLK, 1), lambda rb, kb: (kb, 0)),
            pl.BlockSpec((1, TBLK), lambda rb, kb: (0, rb)),
        ],
        out_specs=[
            pl.BlockSpec((1, 1, TBLK), lambda rb, kb: (rb // TPB, 0, rb % TPB)),
            pl.BlockSpec((1, 1), lambda rb, kb: (0, 0)),
        ],
        out_shape=[
            jax.ShapeDtypeStruct((B, 1, T), jnp.int32),
            jax.ShapeDtypeStruct((1, 1), jnp.float32),
        ],
        scratch_shapes=[
            pltpu.VMEM((1, TBLK), jnp.float32),
            pltpu.VMEM((1, TBLK), jnp.int32),
            pltpu.VMEM((1, TBLK), jnp.float32),
            pltpu.VMEM((1, TBLK), jnp.int32),
            pltpu.VMEM((1, TBLK), jnp.float32),
            pltpu.VMEM((1, TBLK), jnp.int32),
            pltpu.SMEM((1, 1), jnp.float32),
        ],
        compiler_params=pltpu.CompilerParams(
            dimension_semantics=("arbitrary", "arbitrary")),
    )(x_bf, e_bf, e_sq, x_sq)


_NC = 2
_NS = 16
_NW = _NC * _NS
_BPW = N // _NW
_CHUNK = 256
_NCHUNK = _BPW // _CHUNK
_BINS = K // _NW
_NVREG = N // 16


def _sc_gather_hist(table32, idx_flat):
    mesh = plsc.VectorSubcoreMesh(core_axis_name="c", subcore_axis_name="s")

    @functools.partial(
        pl.kernel, mesh=mesh,
        out_type=[
            jax.ShapeDtypeStruct((N, D), jnp.float32),
            jax.ShapeDtypeStruct((K,), jnp.float32),
        ],
        scratch_types=[
            pltpu.VMEM((_CHUNK,), jnp.int32),
            pltpu.VMEM((_CHUNK, D), jnp.float32),
            pltpu.VMEM((N,), jnp.int32),
            pltpu.VMEM((_BINS,), jnp.float32),
            pltpu.SemaphoreType.DMA,
        ],
        compiler_params=pltpu.CompilerParams(needs_layout_passes=False),
    )
    def sc_k(table_hbm, idx_hbm, rows_out, counts_out,
             idx_v, rows_v, idx_all, counts_v, sem):
        wid = lax.axis_index("s") * _NC + lax.axis_index("c")
        base = wid * _BPW
        for c in range(_NCHUNK):
            pltpu.sync_copy(idx_hbm.at[pl.ds(base + c * _CHUNK, _CHUNK)], idx_v)
            pltpu.async_copy(table_hbm.at[idx_v], rows_v, sem).wait()
            pltpu.sync_copy(rows_v, rows_out.at[pl.ds(base + c * _CHUNK, _CHUNK)])
        pltpu.sync_copy(idx_hbm, idx_all)
        for j in range(_BINS // 16):
            counts_v[pl.ds(j * 16, 16)] = jnp.zeros((16,), jnp.float32)
        lo = wid * _BINS

        def body(i, carry):
            v = idx_all[pl.ds(i * 16, 16)]
            m = (v >= lo) & (v < lo + _BINS)
            vl = jnp.where(m, v - lo, 0)
            one = jnp.where(m, 1.0, 0.0).astype(jnp.float32)
            plsc.addupdate_scatter(counts_v, [vl], one)
            return carry

        lax.fori_loop(0, _NVREG, body, 0, unroll=4)
        pltpu.sync_copy(counts_v, counts_out.at[pl.ds(lo, _BINS)])

    return sc_k(table32, idx_flat)


def _k3_body(rows_ref, counts_ref, q_ref, ppl_ref):
    b = pl.program_id(0)
    q_ref[0] = rows_ref[0].T

    @pl.when(b == 0)
    def _():
        p = counts_ref[...] * (1.0 / N)
        s = jnp.sum(p * jnp.log(p + 1e-10))
        ppl_ref[...] = jnp.exp(-s) * jnp.ones((1, 1), jnp.float32)


def _k3(rows3d, counts2d):
    return pl.pallas_call(
        _k3_body,
        grid=(B,),
        in_specs=[
            pl.BlockSpec((1, T, D), lambda b: (b, 0, 0)),
            pl.BlockSpec((1, K), lambda b: (0, 0)),
        ],
        out_specs=[
            pl.BlockSpec((1, D, T), lambda b: (b, 0, 0)),
            pl.BlockSpec((1, 1), lambda b: (0, 0)),
        ],
        out_shape=[
            jax.ShapeDtypeStruct((B, D, T), jnp.float32),
            jax.ShapeDtypeStruct((1, 1), jnp.float32),
        ],
        compiler_params=pltpu.CompilerParams(
            dimension_semantics=("arbitrary",)),
    )(rows3d, counts2d)


def kernel(inputs, embedding):
    x_bf = inputs.astype(jnp.bfloat16)
    e_bf = embedding.astype(jnp.bfloat16)
    table32 = e_bf.astype(jnp.float32)
    x_sq = jnp.sum(inputs ** 2, axis=1).reshape(1, N)
    e_sq = jnp.sum(embedding ** 2, axis=1).reshape(K, 1)

    idx3, loss1 = _k1(x_bf, e_bf, e_sq, x_sq)
    idx_flat = idx3.reshape(N)
    rows, counts = _sc_gather_hist(table32, idx_flat)
    q_st, ppl = _k3(rows.reshape(B, T, D), counts.reshape(1, K))
    return (loss1.reshape(()), q_st, ppl.reshape(()), idx3.reshape(B, T))
